# baseline (speedup 1.0000x reference)
import jax
import jax.numpy as jnp
from jax import lax
from jax.experimental import pallas as pl
from jax.experimental.pallas import tpu as pltpu
from jax.experimental.pallas import tpu_sc as plsc

_BATCH = 16384
_K = 16
_NW = 32
_BPW = _BATCH // _NW
_NGRP = _BPW // _K
_NROW = 2 * _NGRP

_NUSER = 1000000
_TPR = 7813
_TROW = _TPR * 1024
_NTILE = 2 * _TPR
_TPW = (_NTILE + _NW - 1) // _NW
_TAIL0 = 999936


def _tiles_body(wt_hbm, ht_hbm, wtl_hbm, htl_hbm, sem):
  wid = lax.axis_index("s") * 2 + lax.axis_index("c")
  t0 = wid * _TPW
  hi = jnp.minimum(t0 + _TPW, _NTILE)
  lo = jnp.minimum(t0, _NTILE)

  def in_range(t):
    return jnp.logical_and(t >= lo, t < hi)

  n_skip = (in_range(_TPR - 1).astype(jnp.int32)
            + in_range(_NTILE - 1).astype(jnp.int32))
  n_copy = (hi - lo - n_skip) * 2

  def issue(i, carry):
    t = lo + i
    tr = t // _TPR
    tt = t - tr * _TPR

    @pl.when(jnp.logical_and(t < hi, tt < _TPR - 1))
    def _():
      r8 = pl.multiple_of(tr * 8, 8)
      u0 = pl.multiple_of(tt * 128, 128)
      pltpu.async_copy(wt_hbm.at[pl.ds(r8, 8), pl.ds(u0, 128)],
                       wtl_hbm.at[t], sem)
      pltpu.async_copy(ht_hbm.at[pl.ds(r8, 8), pl.ds(u0, 128)],
                       htl_hbm.at[t], sem)
    return carry

  lax.fori_loop(0, _TPW, issue, 0)

  def drain(i, carry):
    pltpu.make_async_copy(wt_hbm.at[pl.ds(0, 8), pl.ds(0, 128)],
                          wtl_hbm.at[0], sem).wait()
    return carry

  lax.fori_loop(0, n_copy, drain, 0)


def _mf_body(wf_hbm, hf_hbm, wtail_hbm, htail_hbm, uidx_hbm, vidx_hbm,
             out_hbm, uidx_v, vidx_v, ugat_v, vgat_v, uadr_v, vadr_v,
             wtail_v, htail_v, out_v, usem, vsem):
  wid = lax.axis_index("s") * 2 + lax.axis_index("c")
  base = wid * _BPW

  pltpu.sync_copy(uidx_hbm.at[pl.ds(base, _BPW)], uidx_v)
  pltpu.sync_copy(vidx_hbm.at[pl.ds(base, _BPW)], vidx_v)
  pltpu.sync_copy(wtail_hbm, wtail_v)
  pltpu.sync_copy(htail_hbm, htail_v)

  def build(g, carry):
    row0 = pl.multiple_of(g * _K, _K)
    u = uidx_v[pl.ds(row0, _K)]
    v = vidx_v[pl.ds(row0, _K)]
    ub = (u >> 7) * 1024 + (u & 127)
    vb = (v >> 7) * 1024 + (v & 127)
    for c in range(_K):
      off = (c // 8) * _TROW + (c % 8) * 128
      r = 2 * g + (c // 8)
      s = (c % 8) * _K
      uadr_v[r, pl.ds(s, _K)] = ub + off
      vadr_v[r, pl.ds(s, _K)] = vb + off
    return carry

  lax.fori_loop(0, _NGRP, build, 0)

  copies = []
  for r in range(_NROW):
    copies.append(pltpu.async_copy(
        wf_hbm.at[uadr_v.at[r]], ugat_v.at[r], usem))
    copies.append(pltpu.async_copy(
        hf_hbm.at[vadr_v.at[r]], vgat_v.at[r], vsem))
  for cp in copies:
    cp.wait()

  def dot(g, carry):
    row0 = pl.multiple_of(g * _K, _K)
    u = uidx_v[pl.ds(row0, _K)]
    v = vidx_v[pl.ds(row0, _K)]
    umask = u >= _TAIL0
    vmask = v >= _TAIL0
    ut = jnp.maximum(u - _TAIL0, 0)
    vt = jnp.maximum(v - _TAIL0, 0)
    acc = jnp.zeros((_K,), jnp.float32)
    for c in range(_K):
      r = 2 * g + (c // 8)
      s = (c % 8) * _K
      cfull = jnp.full((_K,), c, jnp.int32)
      uval = jnp.where(umask, plsc.load_gather(wtail_v, [ut, cfull]),
                       ugat_v[r, pl.ds(s, _K)])
      vval = jnp.where(vmask, plsc.load_gather(htail_v, [vt, cfull]),
                       vgat_v[r, pl.ds(s, _K)])
      acc = acc + uval * vval
    out_v[pl.ds(row0, _K)] = acc
    return carry

  lax.fori_loop(0, _NGRP, dot, 0)

  pltpu.sync_copy(out_v, out_hbm.at[pl.ds(base, _BPW)])


_MESH = dict(core_axis_name="c", subcore_axis_name="s",
             num_cores=2, num_subcores=16)


@jax.jit
def kernel(x, W, H):
  uidx = x[:, 0].astype(jnp.int32)
  vidx = x[:, 1].astype(jnp.int32)

  tiles = pl.kernel(
      _tiles_body,
      out_type=(jax.ShapeDtypeStruct((_NTILE, 8, 128), jnp.float32),
                jax.ShapeDtypeStruct((_NTILE, 8, 128), jnp.float32)),
      mesh=plsc.VectorSubcoreMesh(**_MESH),
      compiler_params=pltpu.CompilerParams(
          needs_layout_passes=False, use_tc_tiling_on_sc=True),
      scratch_types=[pltpu.SemaphoreType.DMA],
  )
  wtl, htl = tiles(W.T, H.T)

  mf = pl.kernel(
      _mf_body,
      out_type=jax.ShapeDtypeStruct((_BATCH,), jnp.float32),
      mesh=plsc.VectorSubcoreMesh(**_MESH),
      compiler_params=pltpu.CompilerParams(needs_layout_passes=False),
      scratch_types=[
          pltpu.VMEM((_BPW,), jnp.int32),
          pltpu.VMEM((_BPW,), jnp.int32),
          pltpu.VMEM((_NROW, 128), jnp.float32),
          pltpu.VMEM((_NROW, 128), jnp.float32),
          pltpu.VMEM((_NROW, 128), jnp.int32),
          pltpu.VMEM((_NROW, 128), jnp.int32),
          pltpu.VMEM((64, _K), jnp.float32),
          pltpu.VMEM((64, _K), jnp.float32),
          pltpu.VMEM((_BPW,), jnp.float32),
          pltpu.SemaphoreType.DMA,
          pltpu.SemaphoreType.DMA,
      ],
  )
  return mf(wtl.reshape(-1), htl.reshape(-1), W[_TAIL0:], H[_TAIL0:],
            uidx, vidx)


# speedup vs baseline: 4.8883x; 4.8883x over previous
import jax
import jax.numpy as jnp
from jax import lax
from jax.experimental import pallas as pl
from jax.experimental.pallas import tpu as pltpu
from jax.experimental.pallas import tpu_sc as plsc

_BATCH = 16384
_K = 16
_NW = 32
_BPW = _BATCH // _NW
_CHUNK = 128
_NCHUNK = _BPW // _CHUNK


def _mf_body(w_hbm, h_hbm, uidx_hbm, vidx_hbm, out_hbm,
             uidx_v, vidx_v, urows_v, vrows_v, out_v, usem, vsem):
  wid = lax.axis_index("s") * 2 + lax.axis_index("c")
  base = wid * _BPW

  pltpu.sync_copy(uidx_hbm.at[pl.ds(wid * _NCHUNK, _NCHUNK)], uidx_v)
  pltpu.sync_copy(vidx_hbm.at[pl.ds(wid * _NCHUNK, _NCHUNK)], vidx_v)

  copies = []
  for j in range(_NCHUNK):
    copies.append(pltpu.async_copy(
        w_hbm.at[uidx_v.at[j]], urows_v.at[pl.ds(j * _CHUNK, _CHUNK)], usem))
    copies.append(pltpu.async_copy(
        h_hbm.at[vidx_v.at[j]], vrows_v.at[pl.ds(j * _CHUNK, _CHUNK)], vsem))
  for c in copies:
    c.wait()

  lanes = lax.iota(jnp.int32, _K)

  def group(g, carry):
    row0 = pl.multiple_of(g * _K, _K)
    rows = row0 + lanes
    acc = jnp.zeros((_K,), jnp.float32)
    for j in range(_K):
      cols = jnp.full((_K,), j, jnp.int32)
      u = plsc.load_gather(urows_v, [rows, cols])
      v = plsc.load_gather(vrows_v, [rows, cols])
      acc = acc + u * v
    out_v[pl.ds(row0, _K)] = acc
    return carry

  lax.fori_loop(0, _BPW // _K, group, 0)

  pltpu.sync_copy(out_v, out_hbm.at[pl.ds(base, _BPW)])


@jax.jit
def kernel(x, W, H):
  uidx = x[:, 0].astype(jnp.int32).reshape(_NW * _NCHUNK, _CHUNK)
  vidx = x[:, 1].astype(jnp.int32).reshape(_NW * _NCHUNK, _CHUNK)

  mf = pl.kernel(
      _mf_body,
      out_type=jax.ShapeDtypeStruct((_BATCH,), jnp.float32),
      mesh=plsc.VectorSubcoreMesh(core_axis_name="c", subcore_axis_name="s",
                                  num_cores=2, num_subcores=16),
      compiler_params=pltpu.CompilerParams(
          needs_layout_passes=False, use_tc_tiling_on_sc=False),
      scratch_types=[
          pltpu.VMEM((_NCHUNK, _CHUNK), jnp.int32),
          pltpu.VMEM((_NCHUNK, _CHUNK), jnp.int32),
          pltpu.VMEM((_BPW, _K), jnp.float32),
          pltpu.VMEM((_BPW, _K), jnp.float32),
          pltpu.VMEM((_BPW,), jnp.float32),
          pltpu.SemaphoreType.DMA,
          pltpu.SemaphoreType.DMA,
      ],
  )
  return mf(W, H, uidx, vidx)
